# baseline (speedup 1.0000x reference)
import functools
import math

import jax
import jax.numpy as jnp
from jax import lax
from jax.experimental import pallas as pl
from jax.experimental.pallas import tpu as pltpu
from jax.experimental.pallas import tpu_sc as plsc

D_MODEL = 64
SCALE = math.sqrt(D_MODEL)
NUM_WORKERS = 32
X_ROWS = 4096
X_COLS = 200
B_SLAB = X_ROWS // NUM_WORKERS
NBUF = 4
LANES = 16


def _make_kernel():
    mesh = plsc.VectorSubcoreMesh(core_axis_name="c", subcore_axis_name="s")

    @functools.partial(
        pl.kernel,
        out_type=jax.ShapeDtypeStruct((X_COLS, D_MODEL, X_ROWS), jnp.float32),
        mesh=mesh,
        compiler_params=pltpu.CompilerParams(
            use_tc_tiling_on_sc=False, needs_layout_passes=False
        ),
        scratch_types=(
            [pltpu.VMEM((X_COLS, B_SLAB), jnp.int32)]
            + [pltpu.VMEM((B_SLAB, D_MODEL), jnp.float32)] * NBUF
            + [pltpu.VMEM((D_MODEL, B_SLAB), jnp.float32)] * NBUF
            + [pltpu.SemaphoreType.DMA] * (2 * NBUF)
        ),
    )
    def gather_scale(xt_hbm, table_hbm, out_hbm, xt_v, *bufs_and_sems):
        rows = list(bufs_and_sems[:NBUF])
        obuf = list(bufs_and_sems[NBUF:2 * NBUF])
        gsem = list(bufs_and_sems[2 * NBUF:3 * NBUF])
        osem = list(bufs_and_sems[3 * NBUF:])
        wid = lax.axis_index("s") * 2 + lax.axis_index("c")
        col0 = wid * B_SLAB

        pltpu.sync_copy(xt_hbm.at[:, pl.ds(col0, B_SLAB)], xt_v)

        def gather_desc(t, b):
            src = table_hbm.at[xt_v.at[t]]
            return pltpu.make_async_copy(src, rows[b], gsem[b])

        def out_desc(t, b):
            dst = out_hbm.at[t, :, pl.ds(col0, B_SLAB)]
            return pltpu.make_async_copy(obuf[b], dst, osem[b])

        for t0 in range(NBUF - 1):
            gather_desc(t0, t0).start()

        def quad_body(q, carry):
            for b in range(NBUF):
                t = q * NBUF + b
                gather_desc(t, b).wait()

                @pl.when(t >= NBUF)
                def _wait_prev_out():
                    out_desc(t - NBUF, b).wait()

                def transpose_body(d, carry2):
                    cols = jnp.full((LANES,), d, dtype=jnp.int32)
                    for j in range(B_SLAB // LANES):
                        rids = lax.iota(jnp.int32, LANES) + (j * LANES)
                        vals = plsc.load_gather(rows[b], [rids, cols])
                        obuf[b][d, pl.ds(j * LANES, LANES)] = vals * SCALE
                    return carry2

                lax.fori_loop(0, D_MODEL, transpose_body, 0, unroll=2)
                out_desc(t, b).start()

                @pl.when(t + NBUF - 1 < X_COLS)
                def _start_next_gather():
                    gather_desc(t + NBUF - 1, (b + NBUF - 1) % NBUF).start()
            return carry

        lax.fori_loop(0, X_COLS // NBUF, quad_body, 0)
        for b in range(NBUF):
            out_desc(X_COLS - NBUF + b, b).wait()

    return gather_scale


_gather_scale = _make_kernel()


def kernel(x, embed):
    out = _gather_scale(x.T, embed)
    return out.transpose((2, 0, 1))


# speedup vs baseline: 1.5474x; 1.5474x over previous
import functools
import math

import jax
import jax.numpy as jnp
from jax import lax
from jax.experimental import pallas as pl
from jax.experimental.pallas import tpu as pltpu
from jax.experimental.pallas import tpu_sc as plsc

D_MODEL = 64
SCALE = math.sqrt(D_MODEL)
NUM_WORKERS = 32
X_ROWS = 4096
X_COLS = 200
B_SLAB = X_ROWS // NUM_WORKERS
NBUF = 4
LANES = 16


def _make_kernel():
    mesh = plsc.VectorSubcoreMesh(core_axis_name="c", subcore_axis_name="s")

    @functools.partial(
        pl.kernel,
        out_type=jax.ShapeDtypeStruct((X_COLS, D_MODEL, X_ROWS), jnp.float32),
        mesh=mesh,
        compiler_params=pltpu.CompilerParams(
            use_tc_tiling_on_sc=False, needs_layout_passes=False
        ),
        scratch_types=(
            [pltpu.VMEM((X_COLS, B_SLAB), jnp.int32)]
            + [pltpu.VMEM((B_SLAB, D_MODEL), jnp.float32)] * NBUF
            + [pltpu.VMEM((D_MODEL, B_SLAB), jnp.float32)] * NBUF
            + [pltpu.SemaphoreType.DMA] * (2 * NBUF)
        ),
    )
    def gather_scale(xt_hbm, table_hbm, out_hbm, xt_v, *bufs_and_sems):
        rows = list(bufs_and_sems[:NBUF])
        obuf = list(bufs_and_sems[NBUF:2 * NBUF])
        gsem = list(bufs_and_sems[2 * NBUF:3 * NBUF])
        osem = list(bufs_and_sems[3 * NBUF:])
        wid = lax.axis_index("s") * 2 + lax.axis_index("c")
        col0 = wid * B_SLAB

        pltpu.sync_copy(xt_hbm.at[:, pl.ds(col0, B_SLAB)], xt_v)

        def gather_desc(t, b):
            src = table_hbm.at[xt_v.at[t]]
            return pltpu.make_async_copy(src, rows[b], gsem[b])

        def out_desc(t, b):
            dst = out_hbm.at[t, :, pl.ds(col0, B_SLAB)]
            return pltpu.make_async_copy(obuf[b], dst, osem[b])

        for t0 in range(NBUF - 1):
            gather_desc(t0, t0).start()

        def quad_body(q, carry):
            for b in range(NBUF):
                t = q * NBUF + b
                gather_desc(t, b).wait()

                @pl.when(t >= NBUF)
                def _wait_prev_out():
                    out_desc(t - NBUF, b).wait()

                def transpose_body(j, carry2):
                    lane = lax.iota(jnp.int32, LANES)
                    rids = lane + j * LANES
                    for k in range(D_MODEL // LANES):
                        for s in range(LANES):
                            perm = lax.rem(lane + s, LANES)
                            dids = perm + k * LANES
                            vals = plsc.load_gather(rows[b], [rids, dids])
                            plsc.store_scatter(obuf[b], [dids, rids],
                                               vals * SCALE)
                    return carry2

                lax.fori_loop(0, B_SLAB // LANES, transpose_body, 0)
                out_desc(t, b).start()

                @pl.when(t + NBUF - 1 < X_COLS)
                def _start_next_gather():
                    gather_desc(t + NBUF - 1, (b + NBUF - 1) % NBUF).start()
            return carry

        lax.fori_loop(0, X_COLS // NBUF, quad_body, 0)
        for b in range(NBUF):
            out_desc(X_COLS - NBUF + b, b).wait()

    return gather_scale


_gather_scale = _make_kernel()


def kernel(x, embed):
    out = _gather_scale(x.T, embed)
    return out.transpose((2, 0, 1))
